# baseline (speedup 1.0000x reference)
import functools

import jax
import jax.numpy as jnp
from jax.experimental import pallas as pl
from jax.experimental.pallas import tpu as pltpu

_FEAT = 128
_NODES = 10000
_BATCH = 8192
_HID = 64


def _lk(v, s=0.01):
    return jax.nn.leaky_relu(v, s)


def _softmax_seg(a, seg, n):
    m = jax.ops.segment_max(a, seg, num_segments=n)
    m = jnp.where(jnp.isfinite(m), m, 0.0)
    e = jnp.exp(a - m[seg])
    s = jax.ops.segment_sum(e, seg, num_segments=n)
    return e / (s[seg] + 1e-16)


def _hgc(x, row, col, eattr, W, att, b, num_nodes, num_edges):
    F = W.shape[1]
    xl = x @ W
    el = eattr @ W
    alpha = xl[row] @ att[:F] + el[col] @ att[F:]
    alpha = jax.nn.leaky_relu(alpha, 0.2)
    alpha = _softmax_seg(alpha, col, num_edges)
    ones = jnp.ones(row.shape[0], dtype=x.dtype)
    D = jax.ops.segment_sum(ones, row, num_segments=num_nodes)
    D = jnp.where(D > 0, 1.0 / D, 0.0)
    Bn = jax.ops.segment_sum(ones, col, num_segments=num_edges)
    Bn = jnp.where(Bn > 0, 1.0 / Bn, 0.0)
    m1 = Bn[col][:, None] * alpha[:, None] * xl[row]
    eo = jax.ops.segment_sum(m1, col, num_segments=num_edges)
    m2 = D[row][:, None] * alpha[:, None] * eo[col]
    return jax.ops.segment_sum(m2, row, num_segments=num_nodes) + b


def _gnorm(x, w, b, ms, eps=1e-5):
    mean = jnp.mean(x, axis=0, keepdims=True)
    out = x - mean * ms
    var = jnp.mean(out * out, axis=0, keepdims=True)
    return w * out / jnp.sqrt(var + eps) + b


_KC = 200


def _att_body(out_ref, w1_ref, b1_ref, w2_ref, av_ref, acc_ref, *, nk):
    k = pl.program_id(0)

    @pl.when(k == 0)
    def _():
        acc_ref[...] = jnp.zeros_like(acc_ref)

    acc_ref[...] += jax.lax.dot_general(
        w1_ref[...], out_ref[...],
        dimension_numbers=(((0,), (0,)), ((), ())),
        preferred_element_type=jnp.float32)

    @pl.when(k == nk - 1)
    def _():
        r = jnp.maximum(acc_ref[...] + b1_ref[...], 0.0)
        av_ref[...] = jax.lax.dot_general(
            r, w2_ref[...],
            dimension_numbers=(((0,), (0,)), ((), ())),
            preferred_element_type=jnp.float32)


def _attention_scores(out, w1, b1, w2):
    n, f = out.shape
    nk = n // _KC
    av = pl.pallas_call(
        functools.partial(_att_body, nk=nk),
        grid=(nk,),
        in_specs=[
            pl.BlockSpec((_KC, f), lambda k: (k, 0)),
            pl.BlockSpec((_KC, n), lambda k: (k, 0)),
            pl.BlockSpec((n, 1), lambda k: (0, 0)),
            pl.BlockSpec((n, 1), lambda k: (0, 0)),
        ],
        out_specs=pl.BlockSpec((f, 1), lambda k: (0, 0)),
        out_shape=jax.ShapeDtypeStruct((f, 1), jnp.float32),
        scratch_shapes=[pltpu.VMEM((n, f), jnp.float32)],
    )(out, w1, b1.reshape(n, 1), w2)
    return av


def kernel(x, edge_index, edge_attr, batch_size, params):
    p = params
    row, col = edge_index[0], edge_index[1]
    N = x.shape[0]
    M = edge_attr.shape[0]

    h = _hgc(x, row, col, edge_attr, p['hgc1_W'], p['hgc1_att'], p['hgc1_b'], N, M)
    h = _lk(_gnorm(h, p['gn1_w'], p['gn1_b'], p['gn1_ms']))
    out1 = _lk(h @ p['fc1_W'] + p['fc1_b'])
    h2 = _hgc(h, row, col, edge_attr, p['hgc2_W'], p['hgc2_att'], p['hgc2_b'], N, M)
    h2 = _lk(_gnorm(h2, p['gn2_w'], p['gn2_b'], p['gn2_ms']))
    out2 = _lk(h2 @ p['fc2_W'] + p['fc2_b'])
    out = jnp.concatenate([x, out1, out2], axis=1)

    av = _attention_scores(out, p['attW1'], p['attb1'], p['attW2'])
    a = jax.nn.sigmoid(av + p['attb2'])
    a = jnp.squeeze(a, -1) - jnp.mean(a)

    H = jax.lax.dynamic_slice_in_dim(out * a[None, :], batch_size - _BATCH, _BATCH, axis=0)
    H1 = _lk(H @ p['cfc1_W'] + p['cfc1_b'])
    distil = H1 @ p['dh_W'] + p['dh_b']
    H2 = _lk(H1 @ p['cfc2_W'] + p['cfc2_b'] + H1)
    logits = H2 @ p['ch_W'] + p['ch_b']
    return logits, distil


# speedup vs baseline: 8.9478x; 8.9478x over previous
import functools

import jax
import jax.numpy as jnp
from jax import lax
from jax.experimental import pallas as pl
from jax.experimental.pallas import tpu as pltpu
from jax.experimental.pallas import tpu_sc as plsc

_F = 128
_N = 10000
_NI = 320000
_BATCH = 8192
_NC = 2
_NS = 16
_L = 16
_C = 80
_NW = _NC * _NS
_PT = _NI // _NW
_NCH = _PT // _C
_ZR = 8
_NCK = _N // _ZR
_KPT = -(-_NCK // _NS)
_NP = 10240
_CC = 128
_NCC = _NP // _CC


def _get_mesh():
    return plsc.VectorSubcoreMesh(
        core_axis_name="c", subcore_axis_name="s",
        num_cores=_NC, num_subcores=_NS)


def _leaky(v, s):
    return jnp.where(v >= 0, v, s * v)


def _sc1_body(acc, xl_hbm, rowc_hbm, colc_hbm, ax_hbm, ae_hbm,
              u_hbm, ev_hbm, eoraw_hbm, cntr_hbm, cntc_hbm, ssum_hbm,
              rowv, colv, evv, gax, gae, cr, cc, sp, rows, uv, zb, sem):
    cid = lax.axis_index("c")
    sid = lax.axis_index("s")
    wid = cid * _NS + sid
    z16 = jnp.zeros((_L,), jnp.float32)

    for i in range(_ZR):
        for k in range(_F // _L):
            zb[i, pl.ds(16 * k, 16)] = z16


    def _zchunk(k, _):
        ck = sid + _NS * k

        @pl.when(ck < _NCK)
        def _():
            off = pl.multiple_of(ck * _ZR, 8)
            pltpu.sync_copy(zb, acc.at[pl.ds(off, _ZR), :])
        return 0
    lax.fori_loop(0, _KPT, _zchunk, 0)

    def _zacc(i, _):
        cr[pl.ds(16 * i, 16)] = z16
        cc[pl.ds(16 * i, 16)] = z16
        sp[pl.ds(16 * i, 16)] = z16
        return 0
    lax.fori_loop(0, _N // _L, _zacc, 0)

    pltpu.sync_copy(u_hbm, uv)
    plsc.subcore_barrier()

    uvec = uv[...]

    def _chunk(j, _):
        pltpu.sync_copy(rowc_hbm.at[wid, j], rowv)
        pltpu.sync_copy(colc_hbm.at[wid, j], colv)
        pltpu.async_copy(xl_hbm.at[rowv], rows, sem).wait()
        pltpu.async_copy(ax_hbm.at[rowv], gax, sem).wait()
        pltpu.async_copy(ae_hbm.at[colv], gae, sem).wait()

        lanes = lax.iota(jnp.int32, _L)
        ones = z16 + 1.0
        for k in range(_C // _L):
            a = gax[pl.ds(16 * k, 16)] + gae[pl.ds(16 * k, 16)]
            a = _leaky(a, 0.2)
            e = jnp.exp(a - uvec)
            evv[pl.ds(16 * k, 16)] = e
            r = rowv[pl.ds(16 * k, 16)]
            cidx = colv[pl.ds(16 * k, 16)]
            for l in range(_L):
                m = lanes == l
                plsc.addupdate_scatter(cr, [r], ones, mask=m)
                plsc.addupdate_scatter(cc, [cidx], ones, mask=m)
                plsc.addupdate_scatter(sp, [cidx], e, mask=m)

        for g in range(_C // _L):
            ev16 = evv[pl.ds(16 * g, 16)]
            for l in range(_L):
                es = ev16[l]
                ri = 16 * g + l
                for k2 in range(_F // _L):
                    rows[ri, pl.ds(16 * k2, 16)] = rows[ri, pl.ds(16 * k2, 16)] * es

        pltpu.sync_copy(evv, ev_hbm.at[wid, j])
        pltpu.sync_copy(rows, acc.at[colv], add=True)
        return 0

    lax.fori_loop(0, _NCH, _chunk, 0)

    pltpu.sync_copy(cr, cntr_hbm.at[wid])
    pltpu.sync_copy(cc, cntc_hbm.at[wid])
    pltpu.sync_copy(sp, ssum_hbm.at[wid])

    plsc.subcore_barrier()

    def _dchunk(k, _):
        ck = sid + _NS * k

        @pl.when(ck < _NCK)
        def _():
            off = pl.multiple_of(ck * _ZR, 8)
            pltpu.sync_copy(acc.at[pl.ds(off, _ZR), :], zb)

            @pl.when(cid == 0)
            def _():
                pltpu.sync_copy(zb, eoraw_hbm.at[0, pl.ds(off, _ZR), :])

            @pl.when(cid == 1)
            def _():
                pltpu.sync_copy(zb, eoraw_hbm.at[1, pl.ds(off, _ZR), :])
        return 0
    lax.fori_loop(0, _KPT, _dchunk, 0)


def _sc1_entry(*refs):
    _sc1_body(refs[-1], *refs[:-1])


def _sc_pass1(xl, rowc, colc, ax, ae, u16):
    f = pl.kernel(
        _sc1_entry,
        out_type=(
            jax.ShapeDtypeStruct((_NW, _NCH, _C), jnp.float32),
            jax.ShapeDtypeStruct((_NC, _N, _F), jnp.float32),
            jax.ShapeDtypeStruct((_NW, _N), jnp.float32),
            jax.ShapeDtypeStruct((_NW, _N), jnp.float32),
            jax.ShapeDtypeStruct((_NW, _N), jnp.float32),
        ),
        mesh=_get_mesh(),
        compiler_params=pltpu.CompilerParams(needs_layout_passes=False),
        scratch_types=[
            pltpu.VMEM((_C,), jnp.int32),
            pltpu.VMEM((_C,), jnp.int32),
            pltpu.VMEM((_C,), jnp.float32),
            pltpu.VMEM((_C,), jnp.float32),
            pltpu.VMEM((_C,), jnp.float32),
            pltpu.VMEM((_N,), jnp.float32),
            pltpu.VMEM((_N,), jnp.float32),
            pltpu.VMEM((_N,), jnp.float32),
            pltpu.VMEM((_C, _F), jnp.float32),
            pltpu.VMEM((_L,), jnp.float32),
            pltpu.VMEM((_ZR, _F), jnp.float32),
            pltpu.SemaphoreType.DMA,
            pltpu.VMEM_SHARED((_N, _F), jnp.float32),
        ],
    )
    return f(xl, rowc, colc, ax, ae, u16)


def _sc2_body(acc, t2_hbm, rowc_hbm, colc_hbm, ev_hbm,
              outraw_hbm, rowv, colv, evv, rows, zb, sem):
    cid = lax.axis_index("c")
    sid = lax.axis_index("s")
    wid = cid * _NS + sid
    z16 = jnp.zeros((_L,), jnp.float32)

    for i in range(_ZR):
        for k in range(_F // _L):
            zb[i, pl.ds(16 * k, 16)] = z16

    def _zchunk(k, _):
        ck = sid + _NS * k

        @pl.when(ck < _NCK)
        def _():
            off = pl.multiple_of(ck * _ZR, 8)
            pltpu.sync_copy(zb, acc.at[pl.ds(off, _ZR), :])
        return 0
    lax.fori_loop(0, _KPT, _zchunk, 0)

    plsc.subcore_barrier()

    def _chunk(j, _):
        pltpu.sync_copy(rowc_hbm.at[wid, j], rowv)
        pltpu.sync_copy(colc_hbm.at[wid, j], colv)
        pltpu.async_copy(t2_hbm.at[colv], rows, sem).wait()
        pltpu.sync_copy(ev_hbm.at[wid, j], evv)

        for g in range(_C // _L):
            ev16 = evv[pl.ds(16 * g, 16)]
            for l in range(_L):
                es = ev16[l]
                ri = 16 * g + l
                for k2 in range(_F // _L):
                    rows[ri, pl.ds(16 * k2, 16)] = rows[ri, pl.ds(16 * k2, 16)] * es

        pltpu.sync_copy(rowc_hbm.at[wid, j], rowv)
        pltpu.sync_copy(rows, acc.at[rowv], add=True)
        return 0

    lax.fori_loop(0, _NCH, _chunk, 0)

    plsc.subcore_barrier()

    def _dchunk(k, _):
        ck = sid + _NS * k

        @pl.when(ck < _NCK)
        def _():
            off = pl.multiple_of(ck * _ZR, 8)
            pltpu.sync_copy(acc.at[pl.ds(off, _ZR), :], zb)

            @pl.when(cid == 0)
            def _():
                pltpu.sync_copy(zb, outraw_hbm.at[0, pl.ds(off, _ZR), :])

            @pl.when(cid == 1)
            def _():
                pltpu.sync_copy(zb, outraw_hbm.at[1, pl.ds(off, _ZR), :])
        return 0
    lax.fori_loop(0, _KPT, _dchunk, 0)


def _sc2_entry(*refs):
    _sc2_body(refs[-1], *refs[:-1])


def _sc_pass2(t2, rowc, colc, ev):
    f = pl.kernel(
        _sc2_entry,
        out_type=jax.ShapeDtypeStruct((_NC, _N, _F), jnp.float32),
        mesh=_get_mesh(),
        compiler_params=pltpu.CompilerParams(needs_layout_passes=False),
        scratch_types=[
            pltpu.VMEM((_C,), jnp.int32),
            pltpu.VMEM((_C,), jnp.int32),
            pltpu.VMEM((_C,), jnp.float32),
            pltpu.VMEM((_C, _F), jnp.float32),
            pltpu.VMEM((_ZR, _F), jnp.float32),
            pltpu.SemaphoreType.DMA,
            pltpu.VMEM_SHARED((_N, _F), jnp.float32),
        ],
    )
    return f(t2, rowc, colc, ev)


def _k1_body(x_ref, ea_ref, w_ref, a1_ref, a2_ref,
             xl_ref, ax_ref, ae_ref, u_ref):
    y1 = jnp.dot(x_ref[...], w_ref[...], preferred_element_type=jnp.float32)
    xl_ref[...] = y1
    ax = jnp.dot(y1, a1_ref[...], preferred_element_type=jnp.float32)
    y2 = jnp.dot(ea_ref[...], w_ref[...], preferred_element_type=jnp.float32)
    ae = jnp.dot(y2, a2_ref[...], preferred_element_type=jnp.float32)
    ax_ref[...] = ax
    ae_ref[...] = ae
    u = _leaky(jnp.max(ax) + jnp.max(ae), 0.2)
    u_ref[...] = jnp.full((1, 1), u, jnp.float32)


def _k1(x, ea, W, att):
    n = x.shape[0]
    return pl.pallas_call(
        _k1_body,
        out_shape=(
            jax.ShapeDtypeStruct((n, _F), jnp.float32),
            jax.ShapeDtypeStruct((n, 1), jnp.float32),
            jax.ShapeDtypeStruct((n, 1), jnp.float32),
            jax.ShapeDtypeStruct((1, 1), jnp.float32),
        ),
    )(x, ea, W, att[:_F].reshape(_F, 1), att[_F:].reshape(_F, 1))


def _k2_body(cntr_ref, cntc_ref, ssum_ref, eoraw_ref, t2_ref, d_ref):
    cnt_r = jnp.sum(cntr_ref[...], axis=0)
    d = jnp.where(cnt_r > 0, 1.0 / cnt_r, 0.0)
    d_ref[...] = d[:, None]
    cnt_c = jnp.sum(cntc_ref[...], axis=0)
    bn = jnp.where(cnt_c > 0, 1.0 / cnt_c, 0.0)
    s = jnp.sum(ssum_ref[...], axis=0) + 1e-16
    coef = (bn / (s * s))[:, None]
    t2_ref[...] = (eoraw_ref[0] + eoraw_ref[1]) * coef


def _k2(cntr, cntc, ssum, eoraw):
    return pl.pallas_call(
        _k2_body,
        out_shape=(
            jax.ShapeDtypeStruct((_N, _F), jnp.float32),
            jax.ShapeDtypeStruct((_N, 1), jnp.float32),
        ),
    )(cntr, cntc, ssum, eoraw)


def _k3_body(outraw_ref, d_ref, b_ref, gw_ref, gb_ref, gms_ref,
             fw_ref, fb_ref, out_ref, h_ref):
    h = (outraw_ref[0] + outraw_ref[1]) * d_ref[...] + b_ref[...]
    mean = jnp.mean(h, axis=0, keepdims=True)
    o = h - mean * gms_ref[...]
    var = jnp.mean(o * o, axis=0, keepdims=True)
    h = gw_ref[...] * o * lax.rsqrt(var + 1e-5) + gb_ref[...]
    h = _leaky(h, 0.01)
    h_ref[...] = h
    out_ref[...] = _leaky(
        jnp.dot(h, fw_ref[...], preferred_element_type=jnp.float32) + fb_ref[...],
        0.01)


def _k3(outraw, d, b, gw, gb, gms, fw, fb):
    return pl.pallas_call(
        _k3_body,
        out_shape=(
            jax.ShapeDtypeStruct((_N, 64), jnp.float32),
            jax.ShapeDtypeStruct((_N, _F), jnp.float32),
        ),
    )(outraw, d, b.reshape(1, _F), gw.reshape(1, _F), gb.reshape(1, _F),
      gms.reshape(1, _F), fw, fb.reshape(1, 64))


_KC = 200


def _att_body(out_ref, w1_ref, b1_ref, w2_ref, av_ref, acc_ref, *, nk):
    k = pl.program_id(0)

    @pl.when(k == 0)
    def _():
        acc_ref[...] = jnp.zeros_like(acc_ref)

    acc_ref[...] += lax.dot_general(
        w1_ref[...], out_ref[...],
        dimension_numbers=(((0,), (0,)), ((), ())),
        preferred_element_type=jnp.float32)

    @pl.when(k == nk - 1)
    def _():
        r = jnp.maximum(acc_ref[...] + b1_ref[...], 0.0)
        av_ref[...] = lax.dot_general(
            r, w2_ref[...],
            dimension_numbers=(((0,), (0,)), ((), ())),
            preferred_element_type=jnp.float32)


def _attention_scores(out, w1, b1, w2):
    n, f = out.shape
    nk = n // _KC
    return pl.pallas_call(
        functools.partial(_att_body, nk=nk),
        grid=(nk,),
        in_specs=[
            pl.BlockSpec((_KC, f), lambda k: (k, 0)),
            pl.BlockSpec((_KC, n), lambda k: (k, 0)),
            pl.BlockSpec((n, 1), lambda k: (0, 0)),
            pl.BlockSpec((n, 1), lambda k: (0, 0)),
        ],
        out_specs=pl.BlockSpec((f, 1), lambda k: (0, 0)),
        out_shape=jax.ShapeDtypeStruct((f, 1), jnp.float32),
        scratch_shapes=[pltpu.VMEM((n, f), jnp.float32)],
    )(out, w1, b1.reshape(n, 1), w2)


_BR = 1024


def _k5_body(h_ref, av_ref, b2_ref, c1w_ref, c1b_ref, dw_ref, db_ref,
             c2w_ref, c2b_ref, cw_ref, cb_ref, logits_ref, distil_ref):
    a = jax.nn.sigmoid(av_ref[...] + b2_ref[...])
    a = a - jnp.mean(a)
    hb = h_ref[...] * a
    h1 = _leaky(
        jnp.dot(hb, c1w_ref[...], preferred_element_type=jnp.float32) + c1b_ref[...],
        0.01)
    distil_ref[...] = jnp.dot(h1, dw_ref[...], preferred_element_type=jnp.float32) + db_ref[...]
    h2 = _leaky(
        jnp.dot(h1, c2w_ref[...], preferred_element_type=jnp.float32) + c2b_ref[...] + h1,
        0.01)
    logits_ref[...] = jnp.dot(h2, cw_ref[...], preferred_element_type=jnp.float32) + cb_ref[...]


def _k5(h, av, p):
    nb = _BATCH // _BR
    f2 = 2 * _F
    od = p['dh_W'].shape[1]
    hid = p['cfc1_W'].shape[1]
    return pl.pallas_call(
        _k5_body,
        grid=(nb,),
        in_specs=[
            pl.BlockSpec((_BR, f2), lambda i: (i, 0)),
            pl.BlockSpec((1, f2), lambda i: (0, 0)),
            pl.BlockSpec((1, 1), lambda i: (0, 0)),
            pl.BlockSpec((f2, hid), lambda i: (0, 0)),
            pl.BlockSpec((1, hid), lambda i: (0, 0)),
            pl.BlockSpec((hid, od), lambda i: (0, 0)),
            pl.BlockSpec((1, od), lambda i: (0, 0)),
            pl.BlockSpec((hid, hid), lambda i: (0, 0)),
            pl.BlockSpec((1, hid), lambda i: (0, 0)),
            pl.BlockSpec((hid, od), lambda i: (0, 0)),
            pl.BlockSpec((1, od), lambda i: (0, 0)),
        ],
        out_specs=(
            pl.BlockSpec((_BR, od), lambda i: (i, 0)),
            pl.BlockSpec((_BR, od), lambda i: (i, 0)),
        ),
        out_shape=(
            jax.ShapeDtypeStruct((_BATCH, od), jnp.float32),
            jax.ShapeDtypeStruct((_BATCH, od), jnp.float32),
        ),
    )(h, av, p['attb2'].reshape(1, 1), p['cfc1_W'], p['cfc1_b'].reshape(1, hid),
      p['dh_W'], p['dh_b'].reshape(1, od), p['cfc2_W'], p['cfc2_b'].reshape(1, hid),
      p['ch_W'], p['ch_b'].reshape(1, od))


def _hgc_layer(xin, ea, rowc, colc, W, att, b, gw, gb, gms, fw, fb):
    xl, ax, ae, u = _k1(xin, ea, W, att)
    u16 = jnp.broadcast_to(u.reshape(1), (_L,))
    ev, eoraw, cntr, cntc, ssum = _sc_pass1(
        xl, rowc, colc, ax.reshape(_N), ae.reshape(_N), u16)
    t2, d = _k2(cntr, cntc, ssum, eoraw)
    outraw = _sc_pass2(t2, rowc, colc, ev)
    out_fc, h = _k3(outraw, d, b, gw, gb, gms, fw, fb)
    return out_fc, h


def kernel(x, edge_index, edge_attr, batch_size, params):
    p = params
    rowc = edge_index[0].reshape(_NW, _NCH, _C)
    colc = edge_index[1].reshape(_NW, _NCH, _C)

    out1, h = _hgc_layer(x, edge_attr, rowc, colc,
                         p['hgc1_W'], p['hgc1_att'], p['hgc1_b'],
                         p['gn1_w'], p['gn1_b'], p['gn1_ms'],
                         p['fc1_W'], p['fc1_b'])
    out2, _ = _hgc_layer(h, edge_attr, rowc, colc,
                         p['hgc2_W'], p['hgc2_att'], p['hgc2_b'],
                         p['gn2_w'], p['gn2_b'], p['gn2_ms'],
                         p['fc2_W'], p['fc2_b'])

    out = jnp.concatenate([x, out1, out2], axis=1)
    av = _attention_scores(out, p['attW1'], p['attb1'], p['attW2'])
    av_row = av.reshape(1, 2 * _F)

    H = lax.dynamic_slice_in_dim(out, batch_size - _BATCH, _BATCH, axis=0)
    logits, distil = _k5(H, av_row, p)
    return logits, distil
